# baseline (speedup 1.0000x reference)
import jax
import jax.numpy as jnp
from jax.experimental import pallas as pl

_BB = 128
_ROWS = _BB * 98


def _probe_body(x_ref, out_ref):
    xb = x_ref[0:32, 0:16]
    out_ref[...] = jnp.zeros((32, 16), jnp.float32) + jnp.sum(xb)


def kernel(patch, conv_w, conv_b, fc_w, fc_b, layer_idx, threshold):
    B, C, H, W = patch.shape
    x = patch.reshape(B * ((C * H * W) // 128), 128)
    return pl.pallas_call(
        _probe_body,
        grid=(B // _BB,),
        in_specs=[pl.BlockSpec((_ROWS, 128), lambda i: (i, 0))],
        out_specs=pl.BlockSpec((B // _BB, 16), lambda i: (0, 0)),
        out_shape=jax.ShapeDtypeStruct((B // _BB, 16), jnp.float32),
    )(x)


# speedup vs baseline: 5.5537x; 5.5537x over previous
import jax
import jax.numpy as jnp
from jax import lax
from jax.experimental import pallas as pl
from jax.experimental.pallas import tpu as pltpu

_TAU = 0.9
_TOP_P = 0.8
_MIN_K = 1
_BB = 128
_Q = 4
_CH = _BB // _Q


def _router_body(x_ref, cwa_ref, cwb_ref, cb_ref, fw_ref, fb_ref, out_ref,
                 buf, sems):
    i = pl.program_id(0)
    nsteps = pl.num_programs(0)

    def start(step, slot):
        base = step * _BB
        for c in range(_Q):
            pltpu.make_async_copy(
                x_ref.at[pl.ds(base + c * _CH, _CH)],
                buf.at[slot, pl.ds(c * _CH, _CH)],
                sems.at[slot, c],
            ).start()

    @pl.when(i == 0)
    def _():
        start(0, 0)

    @pl.when(i + 1 < nsteps)
    def _():
        start(i + 1, (i + 1) % 2)

    slot = i % 2
    for c in range(_Q):
        pltpu.make_async_copy(
            x_ref.at[pl.ds(i * _BB + c * _CH, _CH)],
            buf.at[slot, pl.ds(c * _CH, _CH)],
            sems.at[slot, c],
        ).wait()

    xb = buf[slot]
    dn = (((1,), (0,)), ((), ()))
    ya = lax.dot_general(xb[:, :, 0:64], cwa_ref[...], dn,
                         preferred_element_type=jnp.float32)
    yb = lax.dot_general(xb[:, :, 64:128], cwb_ref[...], dn,
                         preferred_element_type=jnp.float32)
    y = ya + yb
    y = jnp.maximum(y + cb_ref[...][None], 0.0)
    pooled = jnp.mean(y, axis=1)
    logits = (jnp.dot(pooled, fw_ref[...], preferred_element_type=jnp.float32)
              + fb_ref[...])
    s = logits * (1.0 / _TAU)
    s = s - jnp.max(s, axis=-1, keepdims=True)
    e = jnp.exp(s)
    p = e / jnp.sum(e, axis=-1, keepdims=True)
    pi = p[:, :, None]
    pj = p[:, None, :]
    ii = lax.broadcasted_iota(jnp.int32, (_BB, 16, 16), 1)
    jj = lax.broadcasted_iota(jnp.int32, (_BB, 16, 16), 2)
    before = (pj > pi) | ((pj == pi) & (jj <= ii))
    cums = jnp.sum(jnp.where(before, jnp.broadcast_to(pj, before.shape), 0.0),
                   axis=2)
    rank = jnp.sum(before.astype(jnp.int32), axis=2) - 1
    keep = (cums <= _TOP_P) | (rank < _MIN_K)
    masked = jnp.where(keep, p, 0.0)
    denom = jnp.clip(jnp.sum(masked, axis=-1, keepdims=True), 1e-10, None)
    out_ref[...] = masked / denom


def kernel(patch, conv_w, conv_b, fc_w, fc_b, layer_idx, threshold):
    B, C, H, W = patch.shape
    R = (C * H * W) // 128
    x = patch.reshape(B, R, 128)
    return pl.pallas_call(
        _router_body,
        grid=(B // _BB,),
        in_specs=[
            pl.BlockSpec(memory_space=pltpu.MemorySpace.HBM),
            pl.BlockSpec((C // 2, 128), lambda i: (0, 0)),
            pl.BlockSpec((C // 2, 128), lambda i: (0, 0)),
            pl.BlockSpec((1, 128), lambda i: (0, 0)),
            pl.BlockSpec((128, 16), lambda i: (0, 0)),
            pl.BlockSpec((1, 16), lambda i: (0, 0)),
        ],
        out_specs=pl.BlockSpec((_BB, 16), lambda i: (i, 0)),
        out_shape=jax.ShapeDtypeStruct((B, 16), jnp.float32),
        scratch_shapes=[
            pltpu.VMEM((2, _BB, R, 128), jnp.float32),
            pltpu.SemaphoreType.DMA((2, _Q)),
        ],
    )(x, conv_w.T[0::2, :], conv_w.T[1::2, :], conv_b.reshape(1, 128),
      fc_w.T, fc_b.reshape(1, 16))


# speedup vs baseline: 6.2140x; 1.1189x over previous
import functools

import jax
import jax.numpy as jnp
from jax import lax
from jax.experimental import pallas as pl
from jax.experimental.pallas import tpu as pltpu
from jax.experimental.pallas import tpu_sc as plsc

_TAU = 0.9
_TOP_P = 0.8
_MIN_K = 1
_BB = 128
_Q = 4
_CH = _BB // _Q

_NC = 2
_NS = 16
_NW = _NC * _NS


def _gate_body(x_ref, cwa_ref, cwb_ref, cb_ref, fw_ref, fb_ref, out_ref,
               buf, sems):
    i = pl.program_id(0)
    nsteps = pl.num_programs(0)

    def start(step, slot):
        base = step * _BB
        for c in range(_Q):
            pltpu.make_async_copy(
                x_ref.at[pl.ds(base + c * _CH, _CH)],
                buf.at[slot, pl.ds(c * _CH, _CH)],
                sems.at[slot, c],
            ).start()

    @pl.when(i == 0)
    def _():
        start(0, 0)

    @pl.when(i + 1 < nsteps)
    def _():
        start(i + 1, (i + 1) % 2)

    slot = i % 2
    for c in range(_Q):
        pltpu.make_async_copy(
            x_ref.at[pl.ds(i * _BB + c * _CH, _CH)],
            buf.at[slot, pl.ds(c * _CH, _CH)],
            sems.at[slot, c],
        ).wait()

    xb = buf[slot]
    dn = (((1,), (0,)), ((), ()))
    ya = lax.dot_general(xb[:, :, 0:64], cwa_ref[...], dn,
                         preferred_element_type=jnp.float32)
    yb = lax.dot_general(xb[:, :, 64:128], cwb_ref[...], dn,
                         preferred_element_type=jnp.float32)
    y = jnp.maximum(ya + yb + cb_ref[...][None], 0.0)
    pooled = jnp.mean(y, axis=1)
    out_ref[...] = (jnp.dot(pooled, fw_ref[...],
                            preferred_element_type=jnp.float32)
                    + fb_ref[...])


def _gate(patch, conv_w, conv_b, fc_w, fc_b):
    B, C, H, W = patch.shape
    R = (C * H * W) // 128
    x = patch.reshape(B, R, 128)
    return pl.pallas_call(
        _gate_body,
        grid=(B // _BB,),
        in_specs=[
            pl.BlockSpec(memory_space=pltpu.MemorySpace.HBM),
            pl.BlockSpec((C // 2, 128), lambda i: (0, 0)),
            pl.BlockSpec((C // 2, 128), lambda i: (0, 0)),
            pl.BlockSpec((1, 128), lambda i: (0, 0)),
            pl.BlockSpec((128, 16), lambda i: (0, 0)),
            pl.BlockSpec((1, 16), lambda i: (0, 0)),
        ],
        out_specs=pl.BlockSpec((_BB, 16), lambda i: (i, 0)),
        out_shape=jax.ShapeDtypeStruct((B, 16), jnp.float32),
        scratch_shapes=[
            pltpu.VMEM((2, _BB, R, 128), jnp.float32),
            pltpu.SemaphoreType.DMA((2, _Q)),
        ],
    )(x, conv_w.T[0::2, :], conv_w.T[1::2, :], conv_b.reshape(1, 128),
      fc_w.T, fc_b.reshape(1, 16))


def _route_sc(logits):
    B, E = logits.shape
    rpw = B // _NW
    mesh = plsc.VectorSubcoreMesh(core_axis_name="c", subcore_axis_name="s")

    @functools.partial(
        pl.kernel,
        mesh=mesh,
        out_type=jax.ShapeDtypeStruct((B, E), jnp.float32),
        scratch_types=[
            pltpu.VMEM((rpw, E), jnp.float32),
            pltpu.VMEM((rpw, E), jnp.float32),
        ],
        compiler_params=pltpu.CompilerParams(needs_layout_passes=False),
    )
    def route(logits_hbm, out_hbm, rows_v, out_v):
        wid = lax.axis_index("s") * _NC + lax.axis_index("c")
        base = wid * rpw
        pltpu.sync_copy(logits_hbm.at[pl.ds(base, rpw)], rows_v)

        def body(r, carry):
            row = rows_v[r]
            s = row * (1.0 / _TAU)
            e = jnp.exp(s - jnp.max(s))
            p = e / jnp.sum(e)
            lane = lax.iota(jnp.int32, 16)
            sk, sv = plsc.sort_key_val(p, lane, descending=True)
            cs = plsc.cumsum(sk)
            keep = (cs <= _TOP_P) | (lane < _MIN_K)
            masked = jnp.where(keep, sk, 0.0)
            den = jnp.maximum(jnp.sum(masked), 1e-10)
            vals = masked / den
            plsc.store_scatter(out_v,
                               [jnp.full((16,), r, jnp.int32), sv], vals)
            return carry

        lax.fori_loop(0, rpw, body, 0)
        pltpu.sync_copy(out_v, out_hbm.at[pl.ds(base, rpw)])

    return route(logits)


def kernel(patch, conv_w, conv_b, fc_w, fc_b, layer_idx, threshold):
    logits = _gate(patch, conv_w, conv_b, fc_w, fc_b)
    return _route_sc(logits)
